# Initial kernel scaffold; baseline (speedup 1.0000x reference)
import jax
import jax.numpy as jnp
from jax.experimental import pallas as pl


def kernel(up_w, up_b, conv1_w, conv2_w, bn1_gamma, bn1_beta, bn2_gamma, bn2_beta, x1_nchw, x2_nchw):
    raise NotImplementedError("write your pallas kernel here")


# speedup vs baseline: 2.2548x; 2.2548x over previous
import functools

import jax
import jax.numpy as jnp
import numpy as np
from jax.experimental import pallas as pl
from jax.experimental.pallas import tpu as pltpu


def _pack_convt(up_w, W1):
    Cin, Cup = up_w.shape[0], up_w.shape[1]
    A = jnp.transpose(up_w.astype(jnp.float32), (2, 0, 3, 1)).reshape(2, Cin, 2 * Cup)
    eye = jnp.eye(W1, dtype=jnp.float32)
    Wt = jnp.einsum("wu,hij->hwiuj", eye, A)
    return Wt.reshape(2, W1 * Cin, 2 * W1 * Cup)


def _pack_band(w_oihw, W):
    Cin = w_oihw.shape[1]
    Wk = jnp.transpose(w_oihw.astype(jnp.float32), (2, 3, 1, 0))
    mats = []
    for kh in range(3):
        m = None
        for kw in range(3):
            S = jnp.asarray(np.eye(W, k=1 - kw, dtype=np.float32))
            t = jnp.einsum("vw,io->viwo", S, Wk[kh, kw])
            m = t if m is None else m + t
        mats.append(m.reshape(W * Cin, -1))
    return jnp.stack(mats)


def _bn_affine(sum_lane, ssq_lane, count, C, gamma, beta, eps):
    s = jnp.sum(sum_lane.reshape(-1, C), axis=0)
    ss = jnp.sum(ssq_lane.reshape(-1, C), axis=0)
    mean = s / count
    var = ss / count - mean * mean
    scale = gamma / jnp.sqrt(var + eps)
    shift = beta - mean * scale
    return scale, shift


def _k1_body(x1_ref, x2_ref, wt_ref, bt_ref, b1_ref,
             o_ref, s_ref, ss_ref, xcat_ref, *, NB, H1, H2, W2C2):
    P = H2 + 2
    M = NB * P
    Kup = wt_ref.shape[2]

    xcat_ref[...] = jnp.zeros_like(xcat_ref)
    for i in range(NB):
        xcat_ref[pl.ds(i * P + 1, H2), 0:W2C2] = x2_ref[i]

    x1 = x1_ref[...].reshape(NB * H1, x1_ref.shape[2])
    bt = bt_ref[...]
    y0 = jnp.dot(x1, wt_ref[0], preferred_element_type=jnp.float32) + bt
    y1 = jnp.dot(x1, wt_ref[1], preferred_element_type=jnp.float32) + bt
    inter = jnp.stack([y0, y1], axis=1).reshape(NB * 2 * H1, Kup)
    inter = inter.astype(xcat_ref.dtype)
    for i in range(NB):
        xcat_ref[pl.ds(i * P + 1, H2), W2C2:W2C2 + Kup] = inter[i * 2 * H1:(i + 1) * 2 * H1]

    xc = xcat_ref[...]
    acc = jnp.dot(xc[0:M - 2], b1_ref[0], preferred_element_type=jnp.float32)
    acc = acc + jnp.dot(xc[1:M - 1], b1_ref[1], preferred_element_type=jnp.float32)
    acc = acc + jnp.dot(xc[2:M], b1_ref[2], preferred_element_type=jnp.float32)

    p = (jax.lax.broadcasted_iota(jnp.int32, (M - 2, 1), 0) + 1) % P
    mask = ((p >= 1) & (p <= H2)).astype(jnp.float32)
    accm = acc * mask
    s_ref[...] = jnp.sum(accm, axis=0, keepdims=True)
    ss_ref[...] = jnp.sum(accm * acc, axis=0, keepdims=True)

    zrow = jnp.zeros((1, o_ref.shape[1]), jnp.float32)
    o_ref[0:1, :] = zrow
    o_ref[pl.ds(1, M - 2), :] = acc
    o_ref[M - 1:M, :] = zrow


def _k2_body(h_ref, b2_ref, sc_ref, sh_ref,
             o_ref, s_ref, ss_ref, xact_ref, *, NB, H2):
    P = H2 + 2
    M = NB * P

    r = jax.lax.broadcasted_iota(jnp.int32, (M, 1), 0) % P
    valid = (r >= 1) & (r <= H2)
    val = jnp.maximum(h_ref[...] * sc_ref[...] + sh_ref[...], 0.0)
    xact_ref[...] = jnp.where(valid, val, 0.0).astype(xact_ref.dtype)

    xa = xact_ref[...]
    acc = jnp.dot(xa[0:M - 2], b2_ref[0], preferred_element_type=jnp.float32)
    acc = acc + jnp.dot(xa[1:M - 1], b2_ref[1], preferred_element_type=jnp.float32)
    acc = acc + jnp.dot(xa[2:M], b2_ref[2], preferred_element_type=jnp.float32)

    p = (jax.lax.broadcasted_iota(jnp.int32, (M - 2, 1), 0) + 1) % P
    mask = ((p >= 1) & (p <= H2)).astype(jnp.float32)
    accm = acc * mask
    s_ref[...] = jnp.sum(accm, axis=0, keepdims=True)
    ss_ref[...] = jnp.sum(accm * acc, axis=0, keepdims=True)

    for i in range(NB):
        o_ref[i, :, :] = acc[i * P:i * P + H2]


def _k3_body(x_ref, sc_ref, sh_ref, o_ref):
    o_ref[...] = jnp.maximum(x_ref[...] * sc_ref[...] + sh_ref[...], 0.0)


def kernel(up_w, up_b, conv1_w, conv2_w, bn1_gamma, bn1_beta,
           bn2_gamma, bn2_beta, x1_nchw, x2_nchw):
    eps = 1e-5
    x1 = jnp.transpose(x1_nchw, (0, 2, 3, 1)).astype(jnp.float32)
    x2 = jnp.transpose(x2_nchw, (0, 2, 3, 1)).astype(jnp.float32)
    N, H1, W1, Cin = x1.shape
    _, H2, W2, C2 = x2.shape
    Cup = up_w.shape[1]
    Cmid = conv1_w.shape[0]
    Cout = conv2_w.shape[0]
    assert H2 == 2 * H1 and W2 == 2 * W1
    assert conv1_w.shape[1] == C2 + Cup

    bf = jnp.bfloat16
    x1f = x1.reshape(N, H1, W1 * Cin).astype(bf)
    x2f = x2.reshape(N, H2, W2 * C2).astype(bf)

    Wt = _pack_convt(up_w, W1).astype(bf)
    bt = jnp.tile(up_b.astype(jnp.float32), 2 * W1)[None, :]
    B1 = jnp.concatenate([_pack_band(conv1_w[:, :C2], W2),
                          _pack_band(conv1_w[:, C2:], W2)], axis=1).astype(bf)
    B2 = _pack_band(conv2_w, W2).astype(bf)

    W1K = W1 * Cin
    W2C2 = W2 * C2
    Kup = Wt.shape[2]
    WCtot = W2 * (C2 + Cup)
    WCmid = W2 * Cmid
    WCout = W2 * Cout

    NB = 16 if N % 16 == 0 else (8 if N % 8 == 0 else 1)
    G = N // NB
    P = H2 + 2
    M = NB * P

    k1 = functools.partial(_k1_body, NB=NB, H1=H1, H2=H2, W2C2=W2C2)
    h1, s1, ss1 = pl.pallas_call(
        k1,
        out_shape=(
            jax.ShapeDtypeStruct((G * M, WCmid), jnp.float32),
            jax.ShapeDtypeStruct((G, 1, WCmid), jnp.float32),
            jax.ShapeDtypeStruct((G, 1, WCmid), jnp.float32),
        ),
        grid=(G,),
        in_specs=[
            pl.BlockSpec((NB, H1, W1K), lambda g: (g, 0, 0)),
            pl.BlockSpec((NB, H2, W2C2), lambda g: (g, 0, 0)),
            pl.BlockSpec((2, W1K, Kup), lambda g: (0, 0, 0)),
            pl.BlockSpec((1, Kup), lambda g: (0, 0)),
            pl.BlockSpec((3, WCtot, WCmid), lambda g: (0, 0, 0)),
        ],
        out_specs=(
            pl.BlockSpec((M, WCmid), lambda g: (g, 0)),
            pl.BlockSpec((None, 1, WCmid), lambda g: (g, 0, 0)),
            pl.BlockSpec((None, 1, WCmid), lambda g: (g, 0, 0)),
        ),
        scratch_shapes=[pltpu.VMEM((M, WCtot), bf)],
        compiler_params=pltpu.CompilerParams(dimension_semantics=("parallel",)),
    )(x1f, x2f, Wt, bt, B1)

    scale1, shift1 = _bn_affine(s1, ss1, N * H2 * W2, Cmid, bn1_gamma, bn1_beta, eps)

    k2 = functools.partial(_k2_body, NB=NB, H2=H2)
    h2, s2, ss2 = pl.pallas_call(
        k2,
        out_shape=(
            jax.ShapeDtypeStruct((N, H2, WCout), jnp.float32),
            jax.ShapeDtypeStruct((G, 1, WCout), jnp.float32),
            jax.ShapeDtypeStruct((G, 1, WCout), jnp.float32),
        ),
        grid=(G,),
        in_specs=[
            pl.BlockSpec((M, WCmid), lambda g: (g, 0)),
            pl.BlockSpec((3, WCmid, WCout), lambda g: (0, 0, 0)),
            pl.BlockSpec((1, WCmid), lambda g: (0, 0)),
            pl.BlockSpec((1, WCmid), lambda g: (0, 0)),
        ],
        out_specs=(
            pl.BlockSpec((NB, H2, WCout), lambda g: (g, 0, 0)),
            pl.BlockSpec((None, 1, WCout), lambda g: (g, 0, 0)),
            pl.BlockSpec((None, 1, WCout), lambda g: (g, 0, 0)),
        ),
        scratch_shapes=[pltpu.VMEM((M, WCmid), bf)],
        compiler_params=pltpu.CompilerParams(dimension_semantics=("parallel",)),
    )(h1, B2, jnp.tile(scale1, W2)[None, :], jnp.tile(shift1, W2)[None, :])

    scale2, shift2 = _bn_affine(s2, ss2, N * H2 * W2, Cout, bn2_gamma, bn2_beta, eps)

    Mrows = N * H2
    bm = Mrows
    while bm > 512 and bm % 2 == 0:
        bm //= 2
    out = pl.pallas_call(
        _k3_body,
        out_shape=jax.ShapeDtypeStruct((Mrows, WCout), jnp.float32),
        grid=(Mrows // bm,),
        in_specs=[
            pl.BlockSpec((bm, WCout), lambda i: (i, 0)),
            pl.BlockSpec((1, WCout), lambda i: (0, 0)),
            pl.BlockSpec((1, WCout), lambda i: (0, 0)),
        ],
        out_specs=pl.BlockSpec((bm, WCout), lambda i: (i, 0)),
        compiler_params=pltpu.CompilerParams(dimension_semantics=("parallel",)),
    )(h2.reshape(Mrows, WCout), jnp.tile(scale2, W2)[None, :],
      jnp.tile(shift2, W2)[None, :])

    out = out.reshape(N, H2, W2, Cout)
    return jnp.transpose(out, (0, 3, 1, 2))
